# baseline (speedup 1.0000x reference)
import functools

import jax
import jax.numpy as jnp
from jax import lax
from jax.experimental import pallas as pl
from jax.experimental.pallas import tpu as pltpu
from jax.experimental.pallas import tpu_sc as plsc

NC, NS = 2, 16
NW = NC * NS
BATCH, SEQ, EMBED = 1024, 200, 300
EMB_PAD = 304
NIDX = BATCH * SEQ
B_PER_W = NIDX // NW
CHUNK = 128
NCHUNKS = B_PER_W // CHUNK
L = 16


def _sc_gather(xf, Wp):
    mesh = plsc.VectorSubcoreMesh(core_axis_name="c", subcore_axis_name="s")

    @functools.partial(
        pl.kernel,
        out_type=jax.ShapeDtypeStruct((NIDX * EMBED,), jnp.float32),
        mesh=mesh,
        scratch_types=[
            pltpu.VMEM((CHUNK,), jnp.int32),
            pltpu.VMEM((CHUNK, EMB_PAD), jnp.float32),
            pltpu.VMEM((CHUNK * EMBED,), jnp.float32),
            pltpu.SemaphoreType.DMA,
        ],
        compiler_params=pltpu.CompilerParams(use_tc_tiling_on_sc=False),
    )
    def k(x_hbm, w_hbm, out_hbm, idx_v, rows_v, flat_v, sem):
        wid = lax.axis_index("s") * NC + lax.axis_index("c")
        base = wid * B_PER_W

        def body(g, carry):
            off = base + g * CHUNK
            pltpu.sync_copy(x_hbm.at[pl.ds(off, CHUNK)], idx_v)
            pltpu.async_copy(w_hbm.at[idx_v], rows_v, sem).wait()

            def row(r, c2):
                dst = r * EMBED
                for j in range(EMBED // L):
                    flat_v[pl.ds(dst + j * L, L)] = rows_v[r, pl.ds(j * L, L)]
                flat_v[pl.ds(dst + EMBED - L, L)] = rows_v[r, pl.ds(EMBED - L, L)]
                return c2

            lax.fori_loop(0, CHUNK, row, 0)
            pltpu.sync_copy(
                flat_v, out_hbm.at[pl.ds(off * EMBED, CHUNK * EMBED)]
            )
            return carry

        lax.fori_loop(0, NCHUNKS, body, 0)

    return k(xf, Wp)


def kernel(x, W):
    xf = x.reshape(-1)
    Wp = jnp.pad(W, ((0, 0), (0, EMB_PAD - EMBED)))
    out = _sc_gather(xf, Wp)
    return out.reshape(BATCH, SEQ, EMBED)


# speedup vs baseline: 1.2170x; 1.2170x over previous
import functools

import jax
import jax.numpy as jnp
from jax import lax
from jax.experimental import pallas as pl
from jax.experimental.pallas import tpu as pltpu
from jax.experimental.pallas import tpu_sc as plsc

NC, NS = 2, 16
NW = NC * NS
BATCH, SEQ, EMBED = 1024, 200, 300
EMB_PAD = 304
NIDX = BATCH * SEQ
B_PER_W = NIDX // NW
CHUNK = 64
NCHUNKS = B_PER_W // CHUNK
L = 16


def _sc_gather(xf, Wp):
    mesh = plsc.VectorSubcoreMesh(core_axis_name="c", subcore_axis_name="s")

    @functools.partial(
        pl.kernel,
        out_type=jax.ShapeDtypeStruct((NIDX * EMBED,), jnp.float32),
        mesh=mesh,
        scratch_types=[
            pltpu.VMEM((B_PER_W,), jnp.int32),
            pltpu.VMEM((CHUNK, EMB_PAD), jnp.float32),
            pltpu.VMEM((CHUNK, EMB_PAD), jnp.float32),
            pltpu.VMEM((CHUNK * EMBED,), jnp.float32),
            pltpu.VMEM((CHUNK * EMBED,), jnp.float32),
            pltpu.SemaphoreType.DMA,
            pltpu.SemaphoreType.DMA,
            pltpu.SemaphoreType.DMA,
            pltpu.SemaphoreType.DMA,
        ],
        compiler_params=pltpu.CompilerParams(use_tc_tiling_on_sc=False),
    )
    def k(x_hbm, w_hbm, out_hbm, idx_v, rows0, rows1, flat0, flat1,
          gsem0, gsem1, osem0, osem1):
        wid = lax.axis_index("s") * NC + lax.axis_index("c")
        base = wid * B_PER_W
        rows = (rows0, rows1)
        flat = (flat0, flat1)
        gsem = (gsem0, gsem1)
        osem = (osem0, osem1)

        pltpu.sync_copy(x_hbm.at[pl.ds(base, B_PER_W)], idx_v)

        def gather_start(cur, p):
            pltpu.async_copy(
                w_hbm.at[idx_v.at[pl.ds(cur * CHUNK, CHUNK)]], rows[p], gsem[p]
            )

        def gather_wait(p):
            pltpu.make_async_copy(
                w_hbm.at[idx_v.at[pl.ds(0, CHUNK)]], rows[p], gsem[p]
            ).wait()

        def out_start(cur, p):
            pltpu.async_copy(
                flat[p],
                out_hbm.at[pl.ds((base + cur * CHUNK) * EMBED, CHUNK * EMBED)],
                osem[p],
            )

        def out_wait(p):
            pltpu.make_async_copy(
                flat[p], out_hbm.at[pl.ds(0, CHUNK * EMBED)], osem[p]
            ).wait()

        def repack(p):
            def row(r, c2):
                dst = r * EMBED
                for j in range(EMBED // L):
                    flat[p][pl.ds(dst + j * L, L)] = rows[p][r, pl.ds(j * L, L)]
                flat[p][pl.ds(dst + EMBED - L, L)] = (
                    rows[p][r, pl.ds(EMBED - L, L)]
                )
                return c2

            lax.fori_loop(0, CHUNK, row, 0)

        gather_start(0, 0)

        def body(i, carry):
            g = i * 2
            for p in (0, 1):
                cur = g + p

                @pl.when(cur >= 2)
                def _():
                    out_wait(p)

                @pl.when(cur + 1 < NCHUNKS)
                def _():
                    gather_start(cur + 1, p ^ 1)

                gather_wait(p)
                repack(p)
                out_start(cur, p)
            return carry

        lax.fori_loop(0, NCHUNKS // 2, body, 0)
        out_wait(0)
        out_wait(1)

    return k(xf, Wp)


def kernel(x, W):
    xf = x.reshape(-1)
    Wp = jnp.pad(W, ((0, 0), (0, EMB_PAD - EMBED)))
    out = _sc_gather(xf, Wp)
    return out.reshape(BATCH, SEQ, EMBED)


# speedup vs baseline: 2.6122x; 2.1465x over previous
import functools

import jax
import jax.numpy as jnp
from jax import lax
from jax.experimental import pallas as pl
from jax.experimental.pallas import tpu as pltpu
from jax.experimental.pallas import tpu_sc as plsc

NC, NS = 2, 16
NW = NC * NS
BATCH, SEQ, EMBED = 1024, 200, 300
EMB_PAD = 384
NIDX = BATCH * SEQ
B_PER_W = NIDX // NW
CHUNK = 64
NCHUNKS = B_PER_W // CHUNK
L = 16


def _sc_gather(xf, Wp):
    mesh = plsc.VectorSubcoreMesh(core_axis_name="c", subcore_axis_name="s")

    @functools.partial(
        pl.kernel,
        out_type=jax.ShapeDtypeStruct((NIDX, EMBED), jnp.float32),
        mesh=mesh,
        scratch_types=[
            pltpu.VMEM((B_PER_W,), jnp.int32),
            pltpu.VMEM((CHUNK, EMB_PAD), jnp.float32),
            pltpu.VMEM((CHUNK, EMB_PAD), jnp.float32),
            pltpu.VMEM((CHUNK, EMBED), jnp.float32),
            pltpu.VMEM((CHUNK, EMBED), jnp.float32),
            pltpu.SemaphoreType.DMA,
            pltpu.SemaphoreType.DMA,
            pltpu.SemaphoreType.DMA,
            pltpu.SemaphoreType.DMA,
        ],
        compiler_params=pltpu.CompilerParams(use_tc_tiling_on_sc=True),
    )
    def k(x_hbm, w_hbm, out_hbm, idx_v, rows0, rows1, flat0, flat1,
          gsem0, gsem1, osem0, osem1):
        wid = lax.axis_index("s") * NC + lax.axis_index("c")
        base = wid * B_PER_W
        rows = (rows0, rows1)
        flat = (flat0, flat1)
        gsem = (gsem0, gsem1)
        osem = (osem0, osem1)

        pltpu.sync_copy(x_hbm.at[pl.ds(base, B_PER_W)], idx_v)

        def gather_start(cur, p):
            pltpu.async_copy(
                w_hbm.at[idx_v.at[pl.ds(cur * CHUNK, CHUNK)]], rows[p], gsem[p]
            )

        def gather_wait(p):
            pltpu.make_async_copy(
                w_hbm.at[idx_v.at[pl.ds(0, CHUNK)]], rows[p], gsem[p]
            ).wait()

        def out_start(cur, p):
            pltpu.async_copy(
                flat[p], out_hbm.at[pl.ds(base + cur * CHUNK, CHUNK)], osem[p]
            )

        def out_wait(p):
            pltpu.make_async_copy(
                flat[p], out_hbm.at[pl.ds(0, CHUNK)], osem[p]
            ).wait()

        def repack(p):
            def row(r, c2):
                for j in range(EMBED // L):
                    flat[p][r, pl.ds(j * L, L)] = rows[p][r, pl.ds(j * L, L)]
                flat[p][r, pl.ds(EMBED - L, L)] = rows[p][r, pl.ds(EMBED - L, L)]
                return c2

            lax.fori_loop(0, CHUNK, row, 0)

        gather_start(0, 0)

        def body(i, carry):
            g = i * 2
            for p in (0, 1):
                cur = g + p

                @pl.when(cur >= 2)
                def _():
                    out_wait(p)

                @pl.when(cur + 1 < NCHUNKS)
                def _():
                    gather_start(cur + 1, p ^ 1)

                gather_wait(p)
                repack(p)
                out_start(cur, p)
            return carry

        lax.fori_loop(0, NCHUNKS // 2, body, 0)
        out_wait(0)
        out_wait(1)

    return k(xf, Wp)


def kernel(x, W):
    xf = x.reshape(-1)
    Wp = jnp.pad(W, ((0, 0), (0, EMB_PAD - EMBED)))
    out = _sc_gather(xf, Wp)
    return out.reshape(BATCH, SEQ, EMBED)
